# baseline (speedup 1.0000x reference)
import functools
import math

import jax
import jax.numpy as jnp
from jax.experimental import pallas as pl
from jax.experimental.pallas import tpu as pltpu


def _in_proj_kernel(x_ref, w_ref, o_ref, *, bm, n_valid):
    i = pl.program_id(0)
    y = jnp.dot(x_ref[...], w_ref[...], preferred_element_type=jnp.float32)
    rows = i * bm + jax.lax.broadcasted_iota(jnp.int32, y.shape, 0)
    o_ref[...] = jnp.where(rows < n_valid, y, 0.0)


def _agg1_kernel(a_ref, s1_ref, w2_ref, o_ref, acc_ref, *,
                 bm, bk, nk, n_valid, k_last_valid):
    i = pl.program_id(0)
    k = pl.program_id(1)

    @pl.when(k == 0)
    def _():
        acc_ref[...] = jnp.zeros_like(acc_ref)

    a = a_ref[...]
    s1 = s1_ref[pl.ds(pl.multiple_of(k * bk, bk), bk), :]

    @pl.when(k < nk - 1)
    def _():
        acc_ref[...] += jnp.dot(a, s1, preferred_element_type=jnp.float32)

    @pl.when(k == nk - 1)
    def _():
        cols = jax.lax.broadcasted_iota(jnp.int32, a.shape, 1)
        am = jnp.where(cols < k_last_valid, a, 0.0)
        acc = acc_ref[...] + jnp.dot(am, s1, preferred_element_type=jnp.float32)
        x = jnp.maximum(acc, 0.0)
        s2 = jnp.dot(x, w2_ref[...], preferred_element_type=jnp.float32)
        rows = i * bm + jax.lax.broadcasted_iota(jnp.int32, s2.shape, 0)
        o_ref[...] = jnp.where(rows < n_valid, s2, 0.0)


def _agg2_kernel(a_ref, s2_ref, o_ref, *, bk, nk, k_last_valid):
    k = pl.program_id(1)

    @pl.when(k == 0)
    def _():
        o_ref[...] = jnp.zeros_like(o_ref)

    a = a_ref[...]
    s2 = s2_ref[pl.ds(pl.multiple_of(k * bk, bk), bk), :]

    @pl.when(k < nk - 1)
    def _():
        o_ref[...] += jnp.dot(a, s2, preferred_element_type=jnp.float32)

    @pl.when(k == nk - 1)
    def _():
        cols = jax.lax.broadcasted_iota(jnp.int32, a.shape, 1)
        am = jnp.where(cols < k_last_valid, a, 0.0)
        o_ref[...] += jnp.dot(am, s2, preferred_element_type=jnp.float32)


def kernel(inputs, adj, W1, W2):
    n, d_in = inputs.shape
    d_hid = W1.shape[1]
    d_out = W2.shape[1]

    BM = 512
    BK = 1024
    nk = math.ceil(n / BK)
    kp = nk * BK
    ni = math.ceil(n / BM)
    k_last_valid = n - (nk - 1) * BK

    nm1 = kp // BM
    s1 = pl.pallas_call(
        functools.partial(_in_proj_kernel, bm=BM, n_valid=n),
        grid=(nm1,),
        in_specs=[
            pl.BlockSpec((BM, d_in), lambda i: (i, 0)),
            pl.BlockSpec((d_in, d_hid), lambda i: (0, 0)),
        ],
        out_specs=pl.BlockSpec((BM, d_hid), lambda i: (i, 0)),
        out_shape=jax.ShapeDtypeStruct((kp, d_hid), jnp.float32),
        compiler_params=pltpu.CompilerParams(
            dimension_semantics=("parallel",)),
    )(inputs, W1)

    s2 = pl.pallas_call(
        functools.partial(_agg1_kernel, bm=BM, bk=BK, nk=nk,
                          n_valid=n, k_last_valid=k_last_valid),
        grid=(ni, nk),
        in_specs=[
            pl.BlockSpec((BM, BK), lambda i, k: (i, k)),
            pl.BlockSpec((kp, d_hid), lambda i, k: (0, 0)),
            pl.BlockSpec((d_hid, d_out), lambda i, k: (0, 0)),
        ],
        out_specs=pl.BlockSpec((BM, d_out), lambda i, k: (i, 0)),
        out_shape=jax.ShapeDtypeStruct((kp, d_out), jnp.float32),
        scratch_shapes=[pltpu.VMEM((BM, d_hid), jnp.float32)],
        compiler_params=pltpu.CompilerParams(
            dimension_semantics=("parallel", "arbitrary")),
    )(adj, s1, W2)

    out = pl.pallas_call(
        functools.partial(_agg2_kernel, bk=BK, nk=nk,
                          k_last_valid=k_last_valid),
        grid=(ni, nk),
        in_specs=[
            pl.BlockSpec((BM, BK), lambda i, k: (i, k)),
            pl.BlockSpec((kp, d_out), lambda i, k: (0, 0)),
        ],
        out_specs=pl.BlockSpec((BM, d_out), lambda i, k: (i, 0)),
        out_shape=jax.ShapeDtypeStruct((n, d_out), jnp.float32),
        compiler_params=pltpu.CompilerParams(
            dimension_semantics=("parallel", "arbitrary")),
    )(adj, s2)

    return out


# speedup vs baseline: 1.9150x; 1.9150x over previous
import functools
import math

import jax
import jax.numpy as jnp
from jax.experimental import pallas as pl
from jax.experimental.pallas import tpu as pltpu


def _in_proj_kernel(x_ref, w_ref, o_ref):
    o_ref[...] = jnp.dot(x_ref[...], w_ref[...],
                         preferred_element_type=jnp.float32)


def _agg1_kernel(a_ref, s1_ref, w2_ref, o_ref):
    x = jnp.maximum(jnp.dot(a_ref[...], s1_ref[...],
                            preferred_element_type=jnp.float32), 0.0)
    o_ref[...] = jnp.dot(x, w2_ref[...], preferred_element_type=jnp.float32)


def _agg2_kernel(a_ref, s2_ref, o_ref):
    o_ref[...] = jnp.dot(a_ref[...], s2_ref[...],
                         preferred_element_type=jnp.float32)


def kernel(inputs, adj, W1, W2):
    n, d_in = inputs.shape
    d_hid = W1.shape[1]
    d_out = W2.shape[1]

    BM = 400
    ni = math.ceil(n / BM)
    BM1 = 2000
    nm1 = math.ceil(n / BM1)

    s1 = pl.pallas_call(
        _in_proj_kernel,
        grid=(nm1,),
        in_specs=[
            pl.BlockSpec((BM1, d_in), lambda i: (i, 0)),
            pl.BlockSpec((d_in, d_hid), lambda i: (0, 0)),
        ],
        out_specs=pl.BlockSpec((BM1, d_hid), lambda i: (i, 0)),
        out_shape=jax.ShapeDtypeStruct((n, d_hid), jnp.float32),
        compiler_params=pltpu.CompilerParams(
            dimension_semantics=("arbitrary",)),
    )(inputs, W1)

    s2 = pl.pallas_call(
        _agg1_kernel,
        grid=(ni,),
        in_specs=[
            pl.BlockSpec((BM, n), lambda i: (i, 0)),
            pl.BlockSpec((n, d_hid), lambda i: (0, 0)),
            pl.BlockSpec((d_hid, d_out), lambda i: (0, 0)),
        ],
        out_specs=pl.BlockSpec((BM, d_out), lambda i: (i, 0)),
        out_shape=jax.ShapeDtypeStruct((n, d_out), jnp.float32),
        compiler_params=pltpu.CompilerParams(
            dimension_semantics=("arbitrary",)),
    )(adj, s1, W2)

    out = pl.pallas_call(
        _agg2_kernel,
        grid=(ni,),
        in_specs=[
            pl.BlockSpec((BM, n), lambda i: (i, 0)),
            pl.BlockSpec((n, d_out), lambda i: (0, 0)),
        ],
        out_specs=pl.BlockSpec((BM, d_out), lambda i: (i, 0)),
        out_shape=jax.ShapeDtypeStruct((n, d_out), jnp.float32),
        compiler_params=pltpu.CompilerParams(
            dimension_semantics=("arbitrary",)),
    )(adj, s2)

    return out


# speedup vs baseline: 1.9661x; 1.0267x over previous
import functools
import math

import jax
import jax.numpy as jnp
from jax.experimental import pallas as pl
from jax.experimental.pallas import tpu as pltpu


def _agg1_kernel(a_ref, in_ref, w1_ref, w2_ref, o_ref):
    h = jnp.dot(a_ref[...], in_ref[...], preferred_element_type=jnp.float32)
    x = jnp.maximum(jnp.dot(h, w1_ref[...],
                            preferred_element_type=jnp.float32), 0.0)
    o_ref[...] = jnp.dot(x, w2_ref[...], preferred_element_type=jnp.float32)


def _agg2_kernel(a_ref, s2_ref, o_ref):
    o_ref[...] = jnp.dot(a_ref[...], s2_ref[...],
                         preferred_element_type=jnp.float32)


def kernel(inputs, adj, W1, W2):
    n, d_in = inputs.shape
    d_hid = W1.shape[1]
    d_out = W2.shape[1]

    BM = 200
    ni = math.ceil(n / BM)

    s2 = pl.pallas_call(
        _agg1_kernel,
        grid=(ni,),
        in_specs=[
            pl.BlockSpec((BM, n), lambda i: (i, 0)),
            pl.BlockSpec((n, d_in), lambda i: (0, 0)),
            pl.BlockSpec((d_in, d_hid), lambda i: (0, 0)),
            pl.BlockSpec((d_hid, d_out), lambda i: (0, 0)),
        ],
        out_specs=pl.BlockSpec((BM, d_out), lambda i: (i, 0)),
        out_shape=jax.ShapeDtypeStruct((n, d_out), jnp.float32),
        compiler_params=pltpu.CompilerParams(
            dimension_semantics=("arbitrary",)),
    )(adj, inputs, W1, W2)

    out = pl.pallas_call(
        _agg2_kernel,
        grid=(ni,),
        in_specs=[
            pl.BlockSpec((BM, n), lambda i: (i, 0)),
            pl.BlockSpec((n, d_out), lambda i: (0, 0)),
        ],
        out_specs=pl.BlockSpec((BM, d_out), lambda i: (i, 0)),
        out_shape=jax.ShapeDtypeStruct((n, d_out), jnp.float32),
        compiler_params=pltpu.CompilerParams(
            dimension_semantics=("arbitrary",)),
    )(adj, s2)

    return out
